# baseline (speedup 1.0000x reference)
import functools

import jax
import jax.numpy as jnp
from jax import lax
from jax.experimental import pallas as pl
from jax.experimental.pallas import tpu as pltpu
from jax.experimental.pallas import tpu_sc as plsc

NSUB = 16
NC = 2
K = 32
IB = 8
BN = 1024


def _edge_pass(nd, c_chunks):
    rps = nd // NSUB
    assert rps % K == 0
    mesh = plsc.VectorSubcoreMesh(core_axis_name="c", subcore_axis_name="s")
    out_type = [jax.ShapeDtypeStruct((NC, nd, 128), jnp.float32)]
    scratch = [
        pltpu.VMEM((IB, K), jnp.int32),
        pltpu.VMEM((IB, K), jnp.int32),
        pltpu.VMEM((IB, K), jnp.int32),
        pltpu.VMEM((IB, K), jnp.int32),
        pltpu.VMEM((K, 128), jnp.float32),
        pltpu.VMEM((K, 128), jnp.float32),
        pltpu.VMEM((K, 128), jnp.float32),
        pltpu.VMEM((K, 128), jnp.float32),
        pltpu.SemaphoreType.DMA,
        pltpu.VMEM_SHARED((nd, 128), jnp.float32),
    ]

    def body(tab, ia_h, ib_h, i0_h, i1_h, outS, ia_v, ib_v, i0_v, i1_v,
             bufA, bufB, bufL, bufR, sem, S_sh):
        cidx = lax.axis_index("c")
        sid = lax.axis_index("s")
        coff = cidx * 64
        base_row = sid * c_chunks
        zero16 = jnp.zeros((16,), jnp.float32)

        def zrow(i, carry):
            for cc in range(8):
                sl = pl.ds(cc * 16, 16)
                bufL[i, sl] = zero16
                bufR[i, sl] = zero16
            return carry
        lax.fori_loop(0, K, zrow, 0)
        zbase = sid * rps
        for k in range(rps // K):
            pltpu.sync_copy(bufL, S_sh.at[pl.ds(zbase + k * K, K)])
        plsc.subcore_barrier()

        def batch(b, carry):
            brow = base_row + b * IB
            pltpu.sync_copy(ia_h.at[pl.ds(brow, IB)], ia_v)
            pltpu.sync_copy(ib_h.at[pl.ds(brow, IB)], ib_v)
            pltpu.sync_copy(i0_h.at[pl.ds(brow, IB)], i0_v)
            pltpu.sync_copy(i1_h.at[pl.ds(brow, IB)], i1_v)

            def chunk(j, c1):
                pltpu.async_copy(tab.at[ia_v.at[j]], bufA, sem).wait()
                pltpu.async_copy(tab.at[ib_v.at[j]], bufB, sem).wait()

                def elt(r, c2):
                    for cc in range(4):
                        src_sl = pl.ds(coff + cc * 16, 16)
                        v = jnp.maximum(bufA[r, src_sl] + bufB[r, src_sl],
                                        0.0)
                        bufL[r, pl.ds(cc * 16, 16)] = v
                        bufR[r, pl.ds(64 + cc * 16, 16)] = v
                    return c2
                lax.fori_loop(0, K, elt, 0)
                for q in range(K // 16):
                    qs = pl.ds(q * 16, 16)
                    v0 = i0_v[j, qs]
                    v1 = i1_v[j, qs]
                    pltpu.sync_copy(bufL.at[qs], S_sh.at[v0], add=True)
                    pltpu.sync_copy(bufR.at[qs], S_sh.at[v1], add=True)
                return c1
            lax.fori_loop(0, IB, chunk, 0)
            return carry
        lax.fori_loop(0, c_chunks // IB, batch, 0)

        plsc.subcore_barrier()

        for k in range(rps // K):
            sl = pl.ds(zbase + k * K, K)
            pltpu.sync_copy(S_sh.at[sl], outS.at[cidx, sl])

    return pl.kernel(body, out_type=out_type, mesh=mesh,
                     scratch_types=scratch)


def _full_spec(shape):
    return pl.BlockSpec(shape, lambda b: (0,) * len(shape))


def _proj_call(h, wproj, b1cat, nd, hdim):
    def body(h_ref, wp_ref, b1_ref, t_ref):
        p = jnp.dot(h_ref[...], wp_ref[...],
                    preferred_element_type=jnp.float32) + b1_ref[...]
        for q in range(4):
            t_ref[q, :, :] = p[:, q * hdim:(q + 1) * hdim]
    grid = nd // BN
    return pl.pallas_call(
        body,
        grid=(grid,),
        in_specs=[pl.BlockSpec((BN, hdim), lambda b: (b, 0)),
                  _full_spec(wproj.shape), _full_spec(b1cat.shape)],
        out_specs=pl.BlockSpec((4, BN, hdim), lambda b: (0, b, 0)),
        out_shape=jax.ShapeDtypeStruct((4, nd, hdim), jnp.float32),
    )(h, wproj, b1cat)


def _dense_call(h, sc0, sc1, cnt, wm, nd, hdim, out_dim, final):
    def body(h_ref, s0_ref, s1_ref, c_ref, w20a, w20b, w21a, w21b,
             b20, b21, gih, ghh, gbi, gbh, *rest):
        hb = h_ref[...]
        dot = functools.partial(jnp.dot, preferred_element_type=jnp.float32)
        s0 = s0_ref[...]
        s1 = s1_ref[...]
        cb = c_ref[:, 0:1] * b20[...] + c_ref[:, 64:65] * b21[...]
        msg = (dot(s0[:, :64], w20a[...]) + dot(s0[:, 64:], w21a[...]) +
               dot(s1[:, :64], w20b[...]) + dot(s1[:, 64:], w21b[...]) + cb)
        gi = dot(msg, gih[...]) + gbi[...]
        gh = dot(hb, ghh[...]) + gbh[...]
        r = 1.0 / (1.0 + jnp.exp(-(gi[:, :hdim] + gh[:, :hdim])))
        z = 1.0 / (1.0 + jnp.exp(-(gi[:, hdim:2 * hdim] +
                                   gh[:, hdim:2 * hdim])))
        n = jnp.tanh(gi[:, 2 * hdim:] + r * gh[:, 2 * hdim:])
        hn = (1.0 - z) * n + z * hb
        if final:
            w1, bb1, w2, bb2, o_ref = rest
            o = dot(jnp.maximum(dot(hn, w1[...]) + bb1[...], 0.0),
                    w2[...]) + bb2[...]
            o = o - jnp.max(o, axis=1, keepdims=True)
            e = jnp.exp(o)
            o_ref[...] = e / jnp.sum(e, axis=1, keepdims=True)
        else:
            wp_ref, b1c_ref, hn_ref, t_ref = rest
            hn_ref[...] = hn
            p = dot(hn, wp_ref[...]) + b1c_ref[...]
            for q in range(4):
                t_ref[q, :, :] = p[:, q * hdim:(q + 1) * hdim]

    grid = nd // BN
    in_specs = [pl.BlockSpec((BN, hdim), lambda b: (b, 0)),
                pl.BlockSpec((BN, 128), lambda b: (b, 0)),
                pl.BlockSpec((BN, 128), lambda b: (b, 0)),
                pl.BlockSpec((BN, 128), lambda b: (b, 0))]
    in_specs += [_full_spec(w.shape) for w in wm]
    if final:
        out_specs = pl.BlockSpec((BN, out_dim), lambda b: (b, 0))
        out_shape = jax.ShapeDtypeStruct((nd, out_dim), jnp.float32)
    else:
        out_specs = [pl.BlockSpec((BN, hdim), lambda b: (b, 0)),
                     pl.BlockSpec((4, BN, hdim), lambda b: (0, b, 0))]
        out_shape = [jax.ShapeDtypeStruct((nd, hdim), jnp.float32),
                     jax.ShapeDtypeStruct((4, nd, hdim), jnp.float32)]
    return pl.pallas_call(body, grid=(grid,), in_specs=in_specs,
                          out_specs=out_specs, out_shape=out_shape)(
        h, sc0, sc1, cnt, *wm)


def kernel(x, edge_index, edge_type, sfW1, sfb1, sfW2, sfb2, fdW1, fdb1,
           fdW2, fdb2, gWih, gWhh, gbih, gbhh, roW1, rob1, roW2, rob2):
    n, hdim = x.shape
    e = edge_index.shape[1]
    out_dim = roW2.shape[0]
    nd = ((n + 1023) // 1024) * 1024
    if nd == n:
        nd += 1024

    src = edge_index[0].astype(jnp.int32)
    dst = edge_index[1].astype(jnp.int32)
    et = edge_type.astype(jnp.int32)
    cpw = -(-e // (NSUB * K))
    cpw = ((cpw + 7) // 8) * 8
    ep = cpw * NSUB * K
    padn = ep - e
    ga = jnp.concatenate([et * nd + src,
                          jnp.zeros((padn,), jnp.int32)]).reshape(-1, K)
    gb = jnp.concatenate([(2 + et) * nd + dst,
                          jnp.zeros((padn,), jnp.int32)]).reshape(-1, K)
    g0 = jnp.concatenate([jnp.where(et == 0, dst, n),
                          jnp.full((padn,), n, jnp.int32)]).reshape(-1, K)
    g1 = jnp.concatenate([jnp.where(et == 1, dst, n),
                          jnp.full((padn,), n, jnp.int32)]).reshape(-1, K)

    wproj = jnp.concatenate([sfW1[:, :hdim].T, fdW1[:, :hdim].T,
                             sfW1[:, hdim:].T, fdW1[:, hdim:].T], axis=1)
    b1cat = jnp.concatenate([jnp.zeros((2 * hdim,), jnp.float32),
                             sfb1, fdb1]).reshape(1, -1)
    w2t0, w2t1 = sfW2.T, fdW2.T
    wm_common = [w2t0[:64], w2t0[64:], w2t1[:64], w2t1[64:],
                 sfb2.reshape(1, -1), fdb2.reshape(1, -1),
                 gWih.T, gWhh.T, gbih.reshape(1, -1), gbhh.reshape(1, -1)]
    wm_mid = wm_common + [wproj, b1cat]
    wm_fin = wm_common + [roW1.T, rob1.reshape(1, -1),
                          roW2.T, rob2.reshape(1, -1)]

    hpad = jnp.pad(x, ((0, nd - n), (0, 0)))

    edge = _edge_pass(nd, cpw)

    tab_one = jnp.full((4 * nd, hdim), 0.5, jnp.float32)
    (outCnt,) = edge(tab_one, ga, gb, g0, g1)
    cnt = outCnt[0]

    tab = _proj_call(hpad, wproj, b1cat, nd, hdim).reshape(4 * nd, hdim)
    (outS,) = edge(tab, ga, gb, g0, g1)
    h1, t1 = _dense_call(hpad, outS[0], outS[1], cnt,
                         wm_mid, nd, hdim, out_dim, final=False)
    (outS,) = edge(t1.reshape(4 * nd, hdim), ga, gb, g0, g1)
    h2, t2 = _dense_call(h1, outS[0], outS[1], cnt,
                         wm_mid, nd, hdim, out_dim, final=False)
    (outS,) = edge(t2.reshape(4 * nd, hdim), ga, gb, g0, g1)
    out = _dense_call(h2, outS[0], outS[1], cnt,
                      wm_fin, nd, hdim, out_dim, final=True)
    return out[:n]


# speedup vs baseline: 1.2281x; 1.2281x over previous
import functools

import jax
import jax.numpy as jnp
from jax import lax
from jax.experimental import pallas as pl
from jax.experimental.pallas import tpu as pltpu
from jax.experimental.pallas import tpu_sc as plsc

NSUB = 16
NC = 2
K = 32
IB = 8
BN = 1024


def _edge_pass(nd, c_chunks):
    rps = nd // NSUB
    assert rps % K == 0
    mesh = plsc.VectorSubcoreMesh(core_axis_name="c", subcore_axis_name="s")
    out_type = [jax.ShapeDtypeStruct((NC, nd, 128), jnp.float32)]
    scratch = [
        pltpu.VMEM((IB, K), jnp.int32),
        pltpu.VMEM((IB, K), jnp.int32),
        pltpu.VMEM((IB, K), jnp.int32),
        pltpu.VMEM((IB, K), jnp.int32),
        pltpu.VMEM((K, 128), jnp.float32),
        pltpu.VMEM((K, 128), jnp.float32),
        pltpu.VMEM((K, 128), jnp.float32),
        pltpu.VMEM((K, 128), jnp.float32),
        pltpu.SemaphoreType.DMA,
        pltpu.SemaphoreType.DMA,
        pltpu.VMEM_SHARED((nd, 128), jnp.float32),
    ]

    def body(tab, ia_h, ib_h, i0_h, i1_h, outS, ia_v, ib_v, i0_v, i1_v,
             bufA, bufB, bufL, bufR, sem, sem2, S_sh):
        cidx = lax.axis_index("c")
        sid = lax.axis_index("s")
        coff = cidx * 64
        base_row = sid * c_chunks
        zero16 = jnp.zeros((16,), jnp.float32)

        def zrow(i, carry):
            for cc in range(8):
                sl = pl.ds(cc * 16, 16)
                bufL[i, sl] = zero16
                bufR[i, sl] = zero16
            return carry
        lax.fori_loop(0, K, zrow, 0)
        zbase = sid * rps
        for k in range(rps // K):
            pltpu.sync_copy(bufL, S_sh.at[pl.ds(zbase + k * K, K)])
        plsc.subcore_barrier()

        def batch(b, carry):
            brow = base_row + b * IB
            pltpu.sync_copy(ia_h.at[pl.ds(brow, IB)], ia_v)
            pltpu.sync_copy(ib_h.at[pl.ds(brow, IB)], ib_v)
            pltpu.sync_copy(i0_h.at[pl.ds(brow, IB)], i0_v)
            pltpu.sync_copy(i1_h.at[pl.ds(brow, IB)], i1_v)

            def chunk(j, c1):
                d1 = pltpu.async_copy(tab.at[ia_v.at[j]], bufA, sem)
                d2 = pltpu.async_copy(tab.at[ib_v.at[j]], bufB, sem)
                d1.wait()
                d2.wait()

                def elt(r, c2):
                    for cc in range(4):
                        src_sl = pl.ds(coff + cc * 16, 16)
                        v = jnp.maximum(bufA[r, src_sl] + bufB[r, src_sl],
                                        0.0)
                        bufL[r, pl.ds(cc * 16, 16)] = v
                        bufR[r, pl.ds(64 + cc * 16, 16)] = v
                    return c2
                lax.fori_loop(0, K, elt, 0)
                ds = []
                for q in range(K // 16):
                    qs = pl.ds(q * 16, 16)
                    ds.append(pltpu.async_copy(bufL.at[qs],
                                               S_sh.at[i0_v[j, qs]],
                                               sem2, add=True))
                    ds.append(pltpu.async_copy(bufR.at[qs],
                                               S_sh.at[i1_v[j, qs]],
                                               sem2, add=True))
                for d in ds:
                    d.wait()
                return c1
            lax.fori_loop(0, IB, chunk, 0)
            return carry
        lax.fori_loop(0, c_chunks // IB, batch, 0)

        plsc.subcore_barrier()

        for k in range(rps // K):
            sl = pl.ds(zbase + k * K, K)
            pltpu.sync_copy(S_sh.at[sl], outS.at[cidx, sl])

    return pl.kernel(body, out_type=out_type, mesh=mesh,
                     scratch_types=scratch)


def _full_spec(shape):
    return pl.BlockSpec(shape, lambda b: (0,) * len(shape))


def _proj_call(h, wproj, b1cat, nd, hdim):
    def body(h_ref, wp_ref, b1_ref, t_ref):
        p = jnp.dot(h_ref[...], wp_ref[...],
                    preferred_element_type=jnp.float32) + b1_ref[...]
        for q in range(4):
            t_ref[q, :, :] = p[:, q * hdim:(q + 1) * hdim]
    grid = nd // BN
    return pl.pallas_call(
        body,
        grid=(grid,),
        in_specs=[pl.BlockSpec((BN, hdim), lambda b: (b, 0)),
                  _full_spec(wproj.shape), _full_spec(b1cat.shape)],
        out_specs=pl.BlockSpec((4, BN, hdim), lambda b: (0, b, 0)),
        out_shape=jax.ShapeDtypeStruct((4, nd, hdim), jnp.float32),
    )(h, wproj, b1cat)


def _dense_call(h, sc0, sc1, cnt, wm, nd, hdim, out_dim, final):
    def body(h_ref, s0_ref, s1_ref, c_ref, w20a, w20b, w21a, w21b,
             b20, b21, gih, ghh, gbi, gbh, *rest):
        hb = h_ref[...]
        dot = functools.partial(jnp.dot, preferred_element_type=jnp.float32)
        s0 = s0_ref[...]
        s1 = s1_ref[...]
        cb = c_ref[:, 0:1] * b20[...] + c_ref[:, 64:65] * b21[...]
        msg = (dot(s0[:, :64], w20a[...]) + dot(s0[:, 64:], w21a[...]) +
               dot(s1[:, :64], w20b[...]) + dot(s1[:, 64:], w21b[...]) + cb)
        gi = dot(msg, gih[...]) + gbi[...]
        gh = dot(hb, ghh[...]) + gbh[...]
        r = 1.0 / (1.0 + jnp.exp(-(gi[:, :hdim] + gh[:, :hdim])))
        z = 1.0 / (1.0 + jnp.exp(-(gi[:, hdim:2 * hdim] +
                                   gh[:, hdim:2 * hdim])))
        n = jnp.tanh(gi[:, 2 * hdim:] + r * gh[:, 2 * hdim:])
        hn = (1.0 - z) * n + z * hb
        if final:
            w1, bb1, w2, bb2, o_ref = rest
            o = dot(jnp.maximum(dot(hn, w1[...]) + bb1[...], 0.0),
                    w2[...]) + bb2[...]
            o = o - jnp.max(o, axis=1, keepdims=True)
            e = jnp.exp(o)
            o_ref[...] = e / jnp.sum(e, axis=1, keepdims=True)
        else:
            wp_ref, b1c_ref, hn_ref, t_ref = rest
            hn_ref[...] = hn
            p = dot(hn, wp_ref[...]) + b1c_ref[...]
            for q in range(4):
                t_ref[q, :, :] = p[:, q * hdim:(q + 1) * hdim]

    grid = nd // BN
    in_specs = [pl.BlockSpec((BN, hdim), lambda b: (b, 0)),
                pl.BlockSpec((BN, 128), lambda b: (b, 0)),
                pl.BlockSpec((BN, 128), lambda b: (b, 0)),
                pl.BlockSpec((BN, 128), lambda b: (b, 0))]
    in_specs += [_full_spec(w.shape) for w in wm]
    if final:
        out_specs = pl.BlockSpec((BN, out_dim), lambda b: (b, 0))
        out_shape = jax.ShapeDtypeStruct((nd, out_dim), jnp.float32)
    else:
        out_specs = [pl.BlockSpec((BN, hdim), lambda b: (b, 0)),
                     pl.BlockSpec((4, BN, hdim), lambda b: (0, b, 0))]
        out_shape = [jax.ShapeDtypeStruct((nd, hdim), jnp.float32),
                     jax.ShapeDtypeStruct((4, nd, hdim), jnp.float32)]
    return pl.pallas_call(body, grid=(grid,), in_specs=in_specs,
                          out_specs=out_specs, out_shape=out_shape)(
        h, sc0, sc1, cnt, *wm)


def kernel(x, edge_index, edge_type, sfW1, sfb1, sfW2, sfb2, fdW1, fdb1,
           fdW2, fdb2, gWih, gWhh, gbih, gbhh, roW1, rob1, roW2, rob2):
    n, hdim = x.shape
    e = edge_index.shape[1]
    out_dim = roW2.shape[0]
    nd = ((n + 1023) // 1024) * 1024
    if nd == n:
        nd += 1024

    src = edge_index[0].astype(jnp.int32)
    dst = edge_index[1].astype(jnp.int32)
    et = edge_type.astype(jnp.int32)
    cpw = -(-e // (NSUB * K))
    cpw = ((cpw + 7) // 8) * 8
    ep = cpw * NSUB * K
    padn = ep - e
    ga = jnp.concatenate([et * nd + src,
                          jnp.zeros((padn,), jnp.int32)]).reshape(-1, K)
    gb = jnp.concatenate([(2 + et) * nd + dst,
                          jnp.zeros((padn,), jnp.int32)]).reshape(-1, K)
    g0 = jnp.concatenate([jnp.where(et == 0, dst, n),
                          jnp.full((padn,), n, jnp.int32)]).reshape(-1, K)
    g1 = jnp.concatenate([jnp.where(et == 1, dst, n),
                          jnp.full((padn,), n, jnp.int32)]).reshape(-1, K)

    wproj = jnp.concatenate([sfW1[:, :hdim].T, fdW1[:, :hdim].T,
                             sfW1[:, hdim:].T, fdW1[:, hdim:].T], axis=1)
    b1cat = jnp.concatenate([jnp.zeros((2 * hdim,), jnp.float32),
                             sfb1, fdb1]).reshape(1, -1)
    w2t0, w2t1 = sfW2.T, fdW2.T
    wm_common = [w2t0[:64], w2t0[64:], w2t1[:64], w2t1[64:],
                 sfb2.reshape(1, -1), fdb2.reshape(1, -1),
                 gWih.T, gWhh.T, gbih.reshape(1, -1), gbhh.reshape(1, -1)]
    wm_mid = wm_common + [wproj, b1cat]
    wm_fin = wm_common + [roW1.T, rob1.reshape(1, -1),
                          roW2.T, rob2.reshape(1, -1)]

    hpad = jnp.pad(x, ((0, nd - n), (0, 0)))

    edge = _edge_pass(nd, cpw)

    tab_one = jnp.full((4 * nd, hdim), 0.5, jnp.float32)
    (outCnt,) = edge(tab_one, ga, gb, g0, g1)
    cnt = outCnt[0]

    tab = _proj_call(hpad, wproj, b1cat, nd, hdim).reshape(4 * nd, hdim)
    (outS,) = edge(tab, ga, gb, g0, g1)
    h1, t1 = _dense_call(hpad, outS[0], outS[1], cnt,
                         wm_mid, nd, hdim, out_dim, final=False)
    (outS,) = edge(t1.reshape(4 * nd, hdim), ga, gb, g0, g1)
    h2, t2 = _dense_call(h1, outS[0], outS[1], cnt,
                         wm_mid, nd, hdim, out_dim, final=False)
    (outS,) = edge(t2.reshape(4 * nd, hdim), ga, gb, g0, g1)
    out = _dense_call(h2, outS[0], outS[1], cnt,
                      wm_fin, nd, hdim, out_dim, final=True)
    return out[:n]
